# Initial kernel scaffold; baseline (speedup 1.0000x reference)
import jax
import jax.numpy as jnp
from jax.experimental import pallas as pl


def kernel(node_feats, coordinates, edge_index, params):
    raise NotImplementedError("write your pallas kernel here")


# speedup vs baseline: 3.0477x; 3.0477x over previous
import functools

import jax
import jax.numpy as jnp
from jax import lax
from jax.experimental import pallas as pl
from jax.experimental.pallas import tpu as pltpu
from jax.experimental.pallas import tpu_sc as plsc

N = 10000
E = 320000
C = 128

NC = 2
NS = 16
NW = NC * NS
EPW = E // NW
SUB = 80
SUP = 400
NSUB = SUP // SUB
NPAD = 10240
NPT = NPAD // NS

_f32 = jnp.float32
_i32 = jnp.int32


def _tables_tc(nf, w1i, w1j):
    bn = 1000

    def body(nf_ref, wi_ref, wj_ref, ts_ref, tr_ref):
        nfb = nf_ref[...]
        ts_ref[...] = jnp.dot(nfb, wi_ref[...], preferred_element_type=_f32)
        tr_ref[...] = jnp.dot(nfb, wj_ref[...], preferred_element_type=_f32)

    return pl.pallas_call(
        body,
        grid=(N // bn,),
        in_specs=[
            pl.BlockSpec((bn, C), lambda i: (i, 0)),
            pl.BlockSpec((C, C), lambda i: (0, 0)),
            pl.BlockSpec((C, C), lambda i: (0, 0)),
        ],
        out_specs=[pl.BlockSpec((bn, C), lambda i: (i, 0))] * 2,
        out_shape=[jax.ShapeDtypeStruct((N, C), _f32)] * 2,
    )(nf, w1i, w1j)


def _sc_geo(cx_a, cy_a, cz_a, snd, rcv):
    mesh = plsc.VectorSubcoreMesh(core_axis_name="c", subcore_axis_name="s")

    @functools.partial(
        pl.kernel,
        out_type=jax.ShapeDtypeStruct((E * 4,), _f32),
        mesh=mesh,
        scratch_types=(
            pltpu.VMEM((N,), _f32),
            pltpu.VMEM((N,), _f32),
            pltpu.VMEM((N,), _f32),
            pltpu.VMEM((EPW,), _i32),
            pltpu.VMEM((EPW,), _i32),
            pltpu.VMEM((EPW * 4,), _f32),
        ),
        compiler_params=pltpu.CompilerParams(needs_layout_passes=False),
    )
    def k(cx_h, cy_h, cz_h, snd_h, rcv_h, rv_h, cx, cy, cz, ixs, ixr, rbuf):
        wid = lax.axis_index("c") * NS + lax.axis_index("s")
        base = pl.multiple_of(wid * EPW, 8)
        pltpu.sync_copy(cx_h, cx)
        pltpu.sync_copy(cy_h, cy)
        pltpu.sync_copy(cz_h, cz)
        pltpu.sync_copy(snd_h.at[pl.ds(base, EPW)], ixs)
        pltpu.sync_copy(rcv_h.at[pl.ds(base, EPW)], ixr)
        lane = lax.iota(_i32, 16)

        def body(g, carry):
            o16 = pl.multiple_of(g * 16, 8)
            s16 = ixs[pl.ds(o16, 16)]
            r16 = ixr[pl.ds(o16, 16)]
            flat = (g * 64) + lane * 4
            for comp, cref in ((0, cx), (1, cy), (2, cz)):
                ci = plsc.load_gather(cref, [s16])
                cj = plsc.load_gather(cref, [r16])
                plsc.store_scatter(rbuf, [flat + comp], ci - cj)
            return carry

        lax.fori_loop(0, EPW // 16, body, 0)
        pltpu.sync_copy(rbuf, rv_h.at[pl.ds(base * 4, EPW * 4)])

    return k(cx_a, cy_a, cz_a, snd, rcv)


def _sc_gather(ts, tr, snd, rcv):
    mesh = plsc.VectorSubcoreMesh(core_axis_name="c", subcore_axis_name="s")

    @functools.partial(
        pl.kernel,
        out_type=(
            jax.ShapeDtypeStruct((E, C), _f32),
            jax.ShapeDtypeStruct((E, C), _f32),
        ),
        mesh=mesh,
        scratch_types=(
            pltpu.VMEM((EPW,), _i32),
            pltpu.VMEM((EPW,), _i32),
            pltpu.VMEM((SUP, C), _f32),
            pltpu.VMEM((SUP, C), _f32),
            pltpu.SemaphoreType.DMA,
        ),
    )
    def k(ts_h, tr_h, snd_h, rcv_h, ga_h, gb_h, ixs, ixr, rs, rr, sem):
        wid = lax.axis_index("c") * NS + lax.axis_index("s")
        base = pl.multiple_of(wid * EPW, 8)
        pltpu.sync_copy(snd_h.at[pl.ds(base, EPW)], ixs)
        pltpu.sync_copy(rcv_h.at[pl.ds(base, EPW)], ixr)

        def body(g, carry):
            goff = pl.multiple_of(g * SUP, 8)
            cps = []
            for i in range(NSUB):
                isl = pl.ds(pl.multiple_of(goff + i * SUB, 8), SUB)
                bsl = pl.ds(i * SUB, SUB)
                cps.append(pltpu.async_copy(ts_h.at[ixs.at[isl]], rs.at[bsl], sem))
                cps.append(pltpu.async_copy(tr_h.at[ixr.at[isl]], rr.at[bsl], sem))
            for cp in cps:
                cp.wait()
            off = pl.multiple_of(base + goff, 8)
            pltpu.sync_copy(rs, ga_h.at[pl.ds(off, SUP)])
            pltpu.sync_copy(rr, gb_h.at[pl.ds(off, SUP)])
            return carry

        lax.fori_loop(0, EPW // SUP, body, 0)

    return k(ts, tr, snd, rcv)


def _edge_tc(ga, gb, rv, zc16, w1g16, w_abs, b1, w2, b2, wa1, ba1, wa2r, wx1,
             bx1, wx2r, sc2):
    be = 2000

    def body(ga_ref, gb_ref, rv_ref, zc_ref, w1g_ref, wab_ref, b1_ref, w2_ref,
             b2_ref, wa1_ref, ba1_ref, wa2_ref, wx1_ref, bx1_ref, wx2_ref,
             sc_ref, msg_ref, dlt_ref, att_ref):
        r = rv_ref[:, :3]
        a = jnp.sqrt(jnp.sum(r * r, axis=1, keepdims=True))
        rbf16 = jnp.sin(a * zc_ref[...]) / a
        geo = jnp.dot(rbf16, w1g_ref[...], preferred_element_type=_f32)
        geo = geo + a * wab_ref[...]
        h1 = jax.nn.silu(ga_ref[...] + gb_ref[...] + geo + b1_ref[...])
        m = jnp.dot(h1, w2_ref[...], preferred_element_type=_f32) + b2_ref[...]
        ha = jax.nn.silu(jnp.dot(m, wa1_ref[...], preferred_element_type=_f32)
                         + ba1_ref[...])
        logit = jnp.sum(ha * wa2_ref[...], axis=1, keepdims=True) + sc_ref[0]
        att = jax.nn.sigmoid(logit)
        hx = jax.nn.silu(jnp.dot(m, wx1_ref[...], preferred_element_type=_f32)
                         + bx1_ref[...])
        px = jnp.sum(hx * wx2_ref[...], axis=1, keepdims=True) + sc_ref[1]
        delta = r * (px / (a + 1.0))
        msg_ref[...] = m * att
        dlt_ref[...] = jnp.concatenate([delta, jnp.zeros((be, 1), _f32)], axis=1)
        att_ref[...] = att

    full = lambda shape: pl.BlockSpec(shape, lambda i: (0, 0))
    return pl.pallas_call(
        body,
        grid=(E // be,),
        in_specs=[
            pl.BlockSpec((be, C), lambda i: (i, 0)),
            pl.BlockSpec((be, C), lambda i: (i, 0)),
            pl.BlockSpec((be, 4), lambda i: (i, 0)),
            full((1, 16)),
            full((16, C)),
            full((1, C)),
            full((1, C)),
            full((C, C)),
            full((1, C)),
            full((C, C)),
            full((1, C)),
            full((1, C)),
            full((C, C)),
            full((1, C)),
            full((1, C)),
            pl.BlockSpec(memory_space=pltpu.SMEM),
        ],
        out_specs=[
            pl.BlockSpec((be, C), lambda i: (i, 0)),
            pl.BlockSpec((be, 4), lambda i: (i, 0)),
            pl.BlockSpec((be, 1), lambda i: (i, 0)),
        ],
        out_shape=[
            jax.ShapeDtypeStruct((E, C), _f32),
            jax.ShapeDtypeStruct((E, 4), _f32),
            jax.ShapeDtypeStruct((E, 1), _f32),
        ],
    )(ga, gb, rv, zc16, w1g16, w_abs, b1, w2, b2, wa1, ba1, wa2r, wx1, bx1,
      wx2r, sc2)


def _sc_scatter_msg(msg, snd, zrows):
    mesh = plsc.VectorSubcoreMesh(core_axis_name="c", subcore_axis_name="s")

    @functools.partial(
        pl.kernel,
        out_type=jax.ShapeDtypeStruct((NC, NPAD, C), _f32),
        mesh=mesh,
        scratch_types=(
            pltpu.VMEM((SUB,), _i32),
            pltpu.VMEM((SUB, C), _f32),
            pltpu.VMEM_SHARED((NPAD, C), _f32),
        ),
    )
    def k(msg_h, snd_h, z_h, outm_h, ix, rows, acc):
        c = lax.axis_index("c")
        s = lax.axis_index("s")
        wid = c * NS + s
        roff = pl.multiple_of(s * NPT, 8)
        pltpu.sync_copy(z_h.at[pl.ds(roff, NPT)], acc.at[pl.ds(roff, NPT)])
        plsc.subcore_barrier()
        base = pl.multiple_of(wid * EPW, 8)

        def body(g, carry):
            off = pl.multiple_of(base + g * SUB, 8)
            pltpu.sync_copy(snd_h.at[pl.ds(off, SUB)], ix)
            pltpu.sync_copy(msg_h.at[pl.ds(off, SUB)], rows)
            pltpu.sync_copy(rows, acc.at[ix], add=True)
            return carry

        lax.fori_loop(0, EPW // SUB, body, 0)
        plsc.subcore_barrier()
        pltpu.sync_copy(acc.at[pl.ds(roff, NPT)], outm_h.at[c, pl.ds(roff, NPT)])

    return k(msg, snd, zrows)


def _sc_scatter_delta(dvec, snd):
    mesh = plsc.VectorSubcoreMesh(core_axis_name="c", subcore_axis_name="s")

    @functools.partial(
        pl.kernel,
        out_type=jax.ShapeDtypeStruct((NW * N * 4,), _f32),
        mesh=mesh,
        scratch_types=(
            pltpu.VMEM((SUB,), _i32),
            pltpu.VMEM((SUB * 4,), _f32),
            pltpu.VMEM((N * 4,), _f32),
        ),
        compiler_params=pltpu.CompilerParams(needs_layout_passes=False),
    )
    def k(dv_h, snd_h, outd_h, ix, dbuf, dacc):
        c = lax.axis_index("c")
        s = lax.axis_index("s")
        wid = c * NS + s
        z16 = jnp.zeros((16,), _f32)

        def zbody(g, carry):
            dacc[pl.ds(pl.multiple_of(g * 16, 8), 16)] = z16
            return carry

        lax.fori_loop(0, N * 4 // 16, zbody, 0)
        base = pl.multiple_of(wid * EPW, 8)
        lane = lax.iota(_i32, 16)

        def body(g, carry):
            off = pl.multiple_of(base + g * SUB, 8)
            pltpu.sync_copy(snd_h.at[pl.ds(off, SUB)], ix)
            pltpu.sync_copy(dv_h.at[pl.ds(off * 4, SUB * 4)], dbuf)
            for q in range(SUB // 16):
                s16 = ix[pl.ds(q * 16, 16)]
                src = q * 64 + lane * 4
                for comp in range(3):
                    vals = plsc.load_gather(dbuf, [src + comp])
                    plsc.addupdate_scatter(dacc, [s16 * 4 + comp], vals)
            return carry

        lax.fori_loop(0, EPW // SUB, body, 0)
        pltpu.sync_copy(dacc, outd_h.at[pl.ds(pl.multiple_of(wid * N * 4, 8),
                                              N * 4)])

    return k(dvec, snd)


def _node_tc(nf, coords, p0, p1, dparts, wn1a, wn1b, bn1, wn2, bn2):
    bn = 1000

    def body(nf_ref, co_ref, p0_ref, p1_ref, dp_ref, wa_ref, wb_ref, b1_ref,
             w2_ref, b2_ref, nfo_ref, coo_ref):
        m = p0_ref[...] + p1_ref[...]
        delta = jnp.sum(dp_ref[...], axis=0)[:, :3]
        nfb = nf_ref[...]
        h = jax.nn.silu(jnp.dot(nfb, wa_ref[...], preferred_element_type=_f32)
                        + jnp.dot(m, wb_ref[...], preferred_element_type=_f32)
                        + b1_ref[...])
        nfo_ref[...] = jnp.dot(h, w2_ref[...], preferred_element_type=_f32) \
            + b2_ref[...] + nfb
        coo_ref[...] = co_ref[...] + delta

    full = lambda shape: pl.BlockSpec(shape, lambda i: (0, 0))
    return pl.pallas_call(
        body,
        grid=(N // bn,),
        in_specs=[
            pl.BlockSpec((bn, C), lambda i: (i, 0)),
            pl.BlockSpec((bn, 3), lambda i: (i, 0)),
            pl.BlockSpec((bn, C), lambda i: (i, 0)),
            pl.BlockSpec((bn, C), lambda i: (i, 0)),
            pl.BlockSpec((NW, bn, 4), lambda i: (0, i, 0)),
            full((C, C)),
            full((C, C)),
            full((1, C)),
            full((C, C)),
            full((1, C)),
        ],
        out_specs=[
            pl.BlockSpec((bn, C), lambda i: (i, 0)),
            pl.BlockSpec((bn, 3), lambda i: (i, 0)),
        ],
        out_shape=[
            jax.ShapeDtypeStruct((N, C), _f32),
            jax.ShapeDtypeStruct((N, 3), _f32),
        ],
    )(nf, coords, p0, p1, dparts, wn1a, wn1b, bn1, wn2, bn2)


def kernel(node_feats, coordinates, edge_index, params):
    pe, pn, pa, px = params["phi_e"], params["phi_n"], params["att"], params["phi_x"]
    w1 = pe["W1"]
    w1i = w1[:C]
    w1j = w1[C:2 * C]
    w_abs = w1[2 * C:2 * C + 1]
    cut = params["bessel_cut_off"]
    amp = jnp.sqrt(2.0 / cut)
    zc16 = jnp.zeros((1, 16), _f32).at[0, :8].set(params["z_0k"] / cut)
    w1g16 = jnp.zeros((16, C), _f32).at[:8].set(w1[2 * C + 1:] * amp)
    b1 = pe["b1"].reshape(1, C)
    b2 = pe["b2"].reshape(1, C)
    ba1 = pa["b1"].reshape(1, C)
    bx1 = px["b1"].reshape(1, C)
    wa2r = pa["W2"].reshape(1, C)
    wx2r = px["W2"].reshape(1, C)
    sc2 = jnp.concatenate([pa["b2"], px["b2"]])
    wn1 = pn["W1"]
    wn1a, wn1b = wn1[:C], wn1[C:]
    bn1 = pn["b1"].reshape(1, C)
    bn2 = pn["b2"].reshape(1, C)

    snd = edge_index[0]
    rcv = edge_index[1]
    cxyz = coordinates.T

    ts, tr = _tables_tc(node_feats, w1i, w1j)
    rv = _sc_geo(cxyz[0], cxyz[1], cxyz[2], snd, rcv).reshape(E, 4)
    ga, gb = _sc_gather(ts, tr, snd, rcv)
    msg, dvec, att = _edge_tc(ga, gb, rv, zc16, w1g16, w_abs, b1, pe["W2"], b2,
                              pa["W1"], ba1, wa2r, px["W1"], bx1, wx2r, sc2)
    partm = _sc_scatter_msg(msg, snd, jnp.zeros((NPAD, C), _f32))
    partd = _sc_scatter_delta(dvec.reshape(E * 4), snd)
    nf_new, co_new = _node_tc(node_feats, coordinates, partm[0], partm[1],
                              partd.reshape(NW, N, 4), wn1a, wn1b, bn1,
                              pn["W2"], bn2)
    return nf_new, co_new, att


# speedup vs baseline: 3.8219x; 1.2541x over previous
import functools

import jax
import jax.numpy as jnp
from jax import lax
from jax.experimental import pallas as pl
from jax.experimental.pallas import tpu as pltpu
from jax.experimental.pallas import tpu_sc as plsc

N = 10000
E = 320000
C = 128

NC = 2
NS = 16
NW = NC * NS
EPW = E // NW
SUB = 80
SUP = 400
NSUB = SUP // SUB
NPAD = 10240
NPT = NPAD // NS

_f32 = jnp.float32
_i32 = jnp.int32


def _tables_tc(nf, w1i, w1j):
    bn = 1000

    def body(nf_ref, wi_ref, wj_ref, ts_ref, tr_ref):
        nfb = nf_ref[...]
        ts_ref[...] = jnp.dot(nfb, wi_ref[...], preferred_element_type=_f32)
        tr_ref[...] = jnp.dot(nfb, wj_ref[...], preferred_element_type=_f32)

    return pl.pallas_call(
        body,
        grid=(N // bn,),
        in_specs=[
            pl.BlockSpec((bn, C), lambda i: (i, 0)),
            pl.BlockSpec((C, C), lambda i: (0, 0)),
            pl.BlockSpec((C, C), lambda i: (0, 0)),
        ],
        out_specs=[pl.BlockSpec((bn, C), lambda i: (i, 0))] * 2,
        out_shape=[jax.ShapeDtypeStruct((N, C), _f32)] * 2,
    )(nf, w1i, w1j)


def _sc_geo(cx_a, cy_a, cz_a, snd, rcv):
    mesh = plsc.VectorSubcoreMesh(core_axis_name="c", subcore_axis_name="s")

    @functools.partial(
        pl.kernel,
        out_type=jax.ShapeDtypeStruct((E * 4,), _f32),
        mesh=mesh,
        scratch_types=(
            pltpu.VMEM((N,), _f32),
            pltpu.VMEM((N,), _f32),
            pltpu.VMEM((N,), _f32),
            pltpu.VMEM((EPW,), _i32),
            pltpu.VMEM((EPW,), _i32),
            pltpu.VMEM((EPW * 4,), _f32),
        ),
        compiler_params=pltpu.CompilerParams(needs_layout_passes=False),
    )
    def k(cx_h, cy_h, cz_h, snd_h, rcv_h, rv_h, cx, cy, cz, ixs, ixr, rbuf):
        wid = lax.axis_index("c") * NS + lax.axis_index("s")
        base = pl.multiple_of(wid * EPW, 8)
        pltpu.sync_copy(cx_h, cx)
        pltpu.sync_copy(cy_h, cy)
        pltpu.sync_copy(cz_h, cz)
        pltpu.sync_copy(snd_h.at[pl.ds(base, EPW)], ixs)
        pltpu.sync_copy(rcv_h.at[pl.ds(base, EPW)], ixr)
        lane = lax.iota(_i32, 16)

        def body(g, carry):
            o16 = pl.multiple_of(g * 16, 8)
            s16 = ixs[pl.ds(o16, 16)]
            r16 = ixr[pl.ds(o16, 16)]
            flat = (g * 64) + lane * 4
            for comp, cref in ((0, cx), (1, cy), (2, cz)):
                ci = plsc.load_gather(cref, [s16])
                cj = plsc.load_gather(cref, [r16])
                plsc.store_scatter(rbuf, [flat + comp], ci - cj)
            return carry

        lax.fori_loop(0, EPW // 16, body, 0)
        pltpu.sync_copy(rbuf, rv_h.at[pl.ds(base * 4, EPW * 4)])

    return k(cx_a, cy_a, cz_a, snd, rcv)


def _sc_gather(ts, tr, snd, rcv):
    mesh = plsc.VectorSubcoreMesh(core_axis_name="c", subcore_axis_name="s")

    @functools.partial(
        pl.kernel,
        out_type=(
            jax.ShapeDtypeStruct((E, C), _f32),
            jax.ShapeDtypeStruct((E, C), _f32),
        ),
        mesh=mesh,
        scratch_types=(
            pltpu.VMEM((EPW,), _i32),
            pltpu.VMEM((EPW,), _i32),
            pltpu.VMEM((SUP, C), _f32),
            pltpu.VMEM((SUP, C), _f32),
            pltpu.SemaphoreType.DMA,
        ),
    )
    def k(ts_h, tr_h, snd_h, rcv_h, ga_h, gb_h, ixs, ixr, rs, rr, sem):
        wid = lax.axis_index("c") * NS + lax.axis_index("s")
        base = pl.multiple_of(wid * EPW, 8)
        pltpu.sync_copy(snd_h.at[pl.ds(base, EPW)], ixs)
        pltpu.sync_copy(rcv_h.at[pl.ds(base, EPW)], ixr)

        def body(g, carry):
            goff = pl.multiple_of(g * SUP, 8)
            cps = []
            for i in range(NSUB):
                isl = pl.ds(pl.multiple_of(goff + i * SUB, 8), SUB)
                bsl = pl.ds(i * SUB, SUB)
                cps.append(pltpu.async_copy(ts_h.at[ixs.at[isl]], rs.at[bsl], sem))
                cps.append(pltpu.async_copy(tr_h.at[ixr.at[isl]], rr.at[bsl], sem))
            for cp in cps:
                cp.wait()
            off = pl.multiple_of(base + goff, 8)
            pltpu.sync_copy(rs, ga_h.at[pl.ds(off, SUP)])
            pltpu.sync_copy(rr, gb_h.at[pl.ds(off, SUP)])
            return carry

        lax.fori_loop(0, EPW // SUP, body, 0)

    return k(ts, tr, snd, rcv)


_SINCOEF = (6.28318531, -41.34170217, 81.60524536, -76.70576095,
            42.05737007, -15.08455476, 3.77595755, -0.61505996)


def _sin2pi(f):
    f = f - jnp.round(f)
    x2 = f * f
    p = jnp.float32(_SINCOEF[-1])
    for coef in _SINCOEF[-2::-1]:
        p = p * x2 + jnp.float32(coef)
    return f * p


def _edge_tc(ga, gb, rv, zc16, w1g16, w_abs, b1, w2, b2, wax1, bax1, wax2, bax2):
    be = 2000

    def body(ga_ref, gb_ref, rv_ref, zc_ref, w1g_ref, wab_ref, b1_ref, w2_ref,
             b2_ref, wax1_ref, bax1_ref, wax2_ref, bax2_ref,
             msg_ref, dlt_ref, att_ref):
        r = rv_ref[:, :3]
        a = jnp.sqrt(jnp.sum(r * r, axis=1, keepdims=True))
        rbf16 = _sin2pi(a * zc_ref[...]) / a
        geo = jnp.dot(rbf16, w1g_ref[...], preferred_element_type=_f32)
        geo = geo + a * wab_ref[...]
        h1 = jax.nn.silu(ga_ref[...] + gb_ref[...] + geo + b1_ref[...])
        m = jnp.dot(h1, w2_ref[...], preferred_element_type=_f32) + b2_ref[...]
        hax = jax.nn.silu(jnp.dot(m, wax1_ref[...], preferred_element_type=_f32)
                          + bax1_ref[...])
        out2 = jnp.dot(hax, wax2_ref[...], preferred_element_type=_f32) \
            + bax2_ref[...]
        att = jax.nn.sigmoid(out2[:, 0:1])
        px = out2[:, 1:2]
        delta = r * (px / (a + 1.0))
        msg_ref[...] = m * att
        dlt_ref[...] = jnp.concatenate([delta, jnp.zeros((be, 1), _f32)], axis=1)
        att_ref[...] = att

    full = lambda shape: pl.BlockSpec(shape, lambda i: (0, 0))
    return pl.pallas_call(
        body,
        grid=(E // be,),
        in_specs=[
            pl.BlockSpec((be, C), lambda i: (i, 0)),
            pl.BlockSpec((be, C), lambda i: (i, 0)),
            pl.BlockSpec((be, 4), lambda i: (i, 0)),
            full((1, 16)),
            full((16, C)),
            full((1, C)),
            full((1, C)),
            full((C, C)),
            full((1, C)),
            full((C, 2 * C)),
            full((1, 2 * C)),
            full((2 * C, C)),
            full((1, C)),
        ],
        out_specs=[
            pl.BlockSpec((be, C), lambda i: (i, 0)),
            pl.BlockSpec((be, 4), lambda i: (i, 0)),
            pl.BlockSpec((be, 1), lambda i: (i, 0)),
        ],
        out_shape=[
            jax.ShapeDtypeStruct((E, C), _f32),
            jax.ShapeDtypeStruct((E, 4), _f32),
            jax.ShapeDtypeStruct((E, 1), _f32),
        ],
    )(ga, gb, rv, zc16, w1g16, w_abs, b1, w2, b2, wax1, bax1, wax2, bax2)


def _sc_scatter_msg(msg, snd, zrows):
    mesh = plsc.VectorSubcoreMesh(core_axis_name="c", subcore_axis_name="s")

    @functools.partial(
        pl.kernel,
        out_type=jax.ShapeDtypeStruct((NC, NPAD, C), _f32),
        mesh=mesh,
        scratch_types=(
            pltpu.VMEM((SUB,), _i32),
            pltpu.VMEM((SUB, C), _f32),
            pltpu.VMEM_SHARED((NPAD, C), _f32),
        ),
    )
    def k(msg_h, snd_h, z_h, outm_h, ix, rows, acc):
        c = lax.axis_index("c")
        s = lax.axis_index("s")
        wid = c * NS + s
        roff = pl.multiple_of(s * NPT, 8)
        pltpu.sync_copy(z_h.at[pl.ds(roff, NPT)], acc.at[pl.ds(roff, NPT)])
        plsc.subcore_barrier()
        base = pl.multiple_of(wid * EPW, 8)

        def body(g, carry):
            off = pl.multiple_of(base + g * SUB, 8)
            pltpu.sync_copy(snd_h.at[pl.ds(off, SUB)], ix)
            pltpu.sync_copy(msg_h.at[pl.ds(off, SUB)], rows)
            pltpu.sync_copy(rows, acc.at[ix], add=True)
            return carry

        lax.fori_loop(0, EPW // SUB, body, 0)
        plsc.subcore_barrier()
        pltpu.sync_copy(acc.at[pl.ds(roff, NPT)], outm_h.at[c, pl.ds(roff, NPT)])

    return k(msg, snd, zrows)


def _sc_scatter_delta(dvec, snd):
    mesh = plsc.VectorSubcoreMesh(core_axis_name="c", subcore_axis_name="s")

    @functools.partial(
        pl.kernel,
        out_type=jax.ShapeDtypeStruct((NW * N * 4,), _f32),
        mesh=mesh,
        scratch_types=(
            pltpu.VMEM((SUB,), _i32),
            pltpu.VMEM((SUB * 4,), _f32),
            pltpu.VMEM((N * 4,), _f32),
        ),
        compiler_params=pltpu.CompilerParams(needs_layout_passes=False),
    )
    def k(dv_h, snd_h, outd_h, ix, dbuf, dacc):
        c = lax.axis_index("c")
        s = lax.axis_index("s")
        wid = c * NS + s
        z16 = jnp.zeros((16,), _f32)

        def zbody(g, carry):
            dacc[pl.ds(pl.multiple_of(g * 16, 8), 16)] = z16
            return carry

        lax.fori_loop(0, N * 4 // 16, zbody, 0)
        base = pl.multiple_of(wid * EPW, 8)
        lane = lax.iota(_i32, 16)

        def body(g, carry):
            off = pl.multiple_of(base + g * SUB, 8)
            pltpu.sync_copy(snd_h.at[pl.ds(off, SUB)], ix)
            pltpu.sync_copy(dv_h.at[pl.ds(off * 4, SUB * 4)], dbuf)
            for q in range(SUB // 16):
                s16 = ix[pl.ds(q * 16, 16)]
                src = q * 64 + lane * 4
                for comp in range(3):
                    vals = plsc.load_gather(dbuf, [src + comp])
                    plsc.addupdate_scatter(dacc, [s16 * 4 + comp], vals)
            return carry

        lax.fori_loop(0, EPW // SUB, body, 0)
        pltpu.sync_copy(dacc, outd_h.at[pl.ds(pl.multiple_of(wid * N * 4, 8),
                                              N * 4)])

    return k(dvec, snd)


def _node_tc(nf, coords, p0, p1, dparts, wn1a, wn1b, bn1, wn2, bn2):
    bn = 1000

    def body(nf_ref, co_ref, p0_ref, p1_ref, dp_ref, wa_ref, wb_ref, b1_ref,
             w2_ref, b2_ref, nfo_ref, coo_ref):
        m = p0_ref[...] + p1_ref[...]
        delta = jnp.sum(dp_ref[...], axis=0)[:, :3]
        nfb = nf_ref[...]
        h = jax.nn.silu(jnp.dot(nfb, wa_ref[...], preferred_element_type=_f32)
                        + jnp.dot(m, wb_ref[...], preferred_element_type=_f32)
                        + b1_ref[...])
        nfo_ref[...] = jnp.dot(h, w2_ref[...], preferred_element_type=_f32) \
            + b2_ref[...] + nfb
        coo_ref[...] = co_ref[...] + delta

    full = lambda shape: pl.BlockSpec(shape, lambda i: (0, 0))
    return pl.pallas_call(
        body,
        grid=(N // bn,),
        in_specs=[
            pl.BlockSpec((bn, C), lambda i: (i, 0)),
            pl.BlockSpec((bn, 3), lambda i: (i, 0)),
            pl.BlockSpec((bn, C), lambda i: (i, 0)),
            pl.BlockSpec((bn, C), lambda i: (i, 0)),
            pl.BlockSpec((NW, bn, 4), lambda i: (0, i, 0)),
            full((C, C)),
            full((C, C)),
            full((1, C)),
            full((C, C)),
            full((1, C)),
        ],
        out_specs=[
            pl.BlockSpec((bn, C), lambda i: (i, 0)),
            pl.BlockSpec((bn, 3), lambda i: (i, 0)),
        ],
        out_shape=[
            jax.ShapeDtypeStruct((N, C), _f32),
            jax.ShapeDtypeStruct((N, 3), _f32),
        ],
    )(nf, coords, p0, p1, dparts, wn1a, wn1b, bn1, wn2, bn2)


def kernel(node_feats, coordinates, edge_index, params):
    pe, pn, pa, px = params["phi_e"], params["phi_n"], params["att"], params["phi_x"]
    w1 = pe["W1"]
    w1i = w1[:C]
    w1j = w1[C:2 * C]
    w_abs = w1[2 * C:2 * C + 1]
    cut = params["bessel_cut_off"]
    amp = jnp.sqrt(2.0 / cut)
    zc16 = jnp.zeros((1, 16), _f32).at[0, :8].set(
        params["z_0k"] / (2.0 * jnp.pi * cut))
    w1g16 = jnp.zeros((16, C), _f32).at[:8].set(w1[2 * C + 1:] * amp)
    b1 = pe["b1"].reshape(1, C)
    b2 = pe["b2"].reshape(1, C)
    wax1 = jnp.concatenate([pa["W1"], px["W1"]], axis=1)
    bax1 = jnp.concatenate([pa["b1"], px["b1"]]).reshape(1, 2 * C)
    wax2 = jnp.zeros((2 * C, C), _f32)
    wax2 = wax2.at[:C, 0].set(pa["W2"][:, 0]).at[C:, 1].set(px["W2"][:, 0])
    bax2 = jnp.zeros((1, C), _f32)
    bax2 = bax2.at[0, 0].set(pa["b2"][0]).at[0, 1].set(px["b2"][0])
    wn1 = pn["W1"]
    wn1a, wn1b = wn1[:C], wn1[C:]
    bn1 = pn["b1"].reshape(1, C)
    bn2 = pn["b2"].reshape(1, C)

    snd = edge_index[0]
    rcv = edge_index[1]
    cxyz = coordinates.T

    ts, tr = _tables_tc(node_feats, w1i, w1j)
    rv = _sc_geo(cxyz[0], cxyz[1], cxyz[2], snd, rcv).reshape(E, 4)
    ga, gb = _sc_gather(ts, tr, snd, rcv)
    msg, dvec, att = _edge_tc(ga, gb, rv, zc16, w1g16, w_abs, b1, pe["W2"], b2,
                              wax1, bax1, wax2, bax2)
    partm = _sc_scatter_msg(msg, snd, jnp.zeros((NPAD, C), _f32))
    partd = _sc_scatter_delta(dvec.reshape(E * 4), snd)
    nf_new, co_new = _node_tc(node_feats, coordinates, partm[0], partm[1],
                              partd.reshape(NW, N, 4), wn1a, wn1b, bn1,
                              pn["W2"], bn2)
    return nf_new, co_new, att


# speedup vs baseline: 4.4200x; 1.1565x over previous
import functools

import jax
import jax.numpy as jnp
from jax import lax
from jax.experimental import pallas as pl
from jax.experimental.pallas import tpu as pltpu
from jax.experimental.pallas import tpu_sc as plsc

N = 10000
E = 320000
C = 128

NC = 2
NS = 16
NW = NC * NS
EPW = E // NW
SUB = 80
SUP = 400
NSUB = SUP // SUB
NPAD = 10240
NPT = NPAD // NS

_f32 = jnp.float32
_bf16 = jnp.bfloat16
_i32 = jnp.int32


def _tables_tc(nf, w1i, w1j):
    bn = 1000

    def body(nf_ref, wi_ref, wj_ref, ts_ref, tr_ref):
        nfb = nf_ref[...]
        ts_ref[...] = jnp.dot(nfb, wi_ref[...], preferred_element_type=_f32)
        tr_ref[...] = jnp.dot(nfb, wj_ref[...], preferred_element_type=_f32)

    return pl.pallas_call(
        body,
        grid=(N // bn,),
        in_specs=[
            pl.BlockSpec((bn, C), lambda i: (i, 0)),
            pl.BlockSpec((C, C), lambda i: (0, 0)),
            pl.BlockSpec((C, C), lambda i: (0, 0)),
        ],
        out_specs=[pl.BlockSpec((bn, C), lambda i: (i, 0))] * 2,
        out_shape=[jax.ShapeDtypeStruct((N, C), _f32)] * 2,
    )(nf, w1i, w1j)


def _sc_geo(cx_a, cy_a, cz_a, snd, rcv):
    mesh = plsc.VectorSubcoreMesh(core_axis_name="c", subcore_axis_name="s")

    @functools.partial(
        pl.kernel,
        out_type=jax.ShapeDtypeStruct((E * 4,), _f32),
        mesh=mesh,
        scratch_types=(
            pltpu.VMEM((N,), _f32),
            pltpu.VMEM((N,), _f32),
            pltpu.VMEM((N,), _f32),
            pltpu.VMEM((EPW,), _i32),
            pltpu.VMEM((EPW,), _i32),
            pltpu.VMEM((EPW * 4,), _f32),
        ),
        compiler_params=pltpu.CompilerParams(needs_layout_passes=False),
    )
    def k(cx_h, cy_h, cz_h, snd_h, rcv_h, rv_h, cx, cy, cz, ixs, ixr, rbuf):
        wid = lax.axis_index("c") * NS + lax.axis_index("s")
        base = pl.multiple_of(wid * EPW, 8)
        pltpu.sync_copy(cx_h, cx)
        pltpu.sync_copy(cy_h, cy)
        pltpu.sync_copy(cz_h, cz)
        pltpu.sync_copy(snd_h.at[pl.ds(base, EPW)], ixs)
        pltpu.sync_copy(rcv_h.at[pl.ds(base, EPW)], ixr)
        lane = lax.iota(_i32, 16)

        def body(g, carry):
            o16 = pl.multiple_of(g * 16, 8)
            s16 = ixs[pl.ds(o16, 16)]
            r16 = ixr[pl.ds(o16, 16)]
            flat = (g * 64) + lane * 4
            for comp, cref in ((0, cx), (1, cy), (2, cz)):
                ci = plsc.load_gather(cref, [s16])
                cj = plsc.load_gather(cref, [r16])
                plsc.store_scatter(rbuf, [flat + comp], ci - cj)
            return carry

        lax.fori_loop(0, EPW // 16, body, 0)
        pltpu.sync_copy(rbuf, rv_h.at[pl.ds(base * 4, EPW * 4)])

    return k(cx_a, cy_a, cz_a, snd, rcv)


NCH = EPW // SUB


def _sc_gather(ts, tr, snd, rcv):
    mesh = plsc.VectorSubcoreMesh(core_axis_name="c", subcore_axis_name="s")

    @functools.partial(
        pl.kernel,
        out_type=(
            jax.ShapeDtypeStruct((E, C), _f32),
            jax.ShapeDtypeStruct((E, C), _f32),
        ),
        mesh=mesh,
        scratch_types=(
            pltpu.VMEM((EPW,), _i32),
            pltpu.VMEM((EPW,), _i32),
            pltpu.VMEM((SUB, C), _f32),
            pltpu.VMEM((SUB, C), _f32),
            pltpu.VMEM((SUB, C), _f32),
            pltpu.VMEM((SUB, C), _f32),
            pltpu.SemaphoreType.DMA,
            pltpu.SemaphoreType.DMA,
        ),
    )
    def k(ts_h, tr_h, snd_h, rcv_h, ga_h, gb_h, ixs, ixr, rs0, rr0, rs1, rr1,
          sem0, sem1):
        wid = lax.axis_index("c") * NS + lax.axis_index("s")
        base = pl.multiple_of(wid * EPW, 8)
        pltpu.sync_copy(snd_h.at[pl.ds(base, EPW)], ixs)
        pltpu.sync_copy(rcv_h.at[pl.ds(base, EPW)], ixr)

        def fire(g, rs, rr, sem):
            isl = pl.ds(pl.multiple_of(g * SUB, 8), SUB)
            pltpu.async_copy(ts_h.at[ixs.at[isl]], rs, sem)
            pltpu.async_copy(tr_h.at[ixr.at[isl]], rr, sem)

        def drain_store(g, rs, rr, sem):
            isl = pl.ds(pl.multiple_of(g * SUB, 8), SUB)
            pltpu.make_async_copy(ts_h.at[ixs.at[isl]], rs, sem).wait()
            pltpu.make_async_copy(tr_h.at[ixr.at[isl]], rr, sem).wait()
            off = pl.multiple_of(base + g * SUB, 8)
            pltpu.sync_copy(rs, ga_h.at[pl.ds(off, SUB)])
            pltpu.sync_copy(rr, gb_h.at[pl.ds(off, SUB)])

        fire(0, rs0, rr0, sem0)

        def body(h, carry):
            g = h * 2
            fire(g + 1, rs1, rr1, sem1)
            drain_store(g, rs0, rr0, sem0)
            fire(g + 2, rs0, rr0, sem0)
            drain_store(g + 1, rs1, rr1, sem1)
            return carry

        lax.fori_loop(0, (NCH - 1) // 2, body, 0)
        drain_store(NCH - 1, rs0, rr0, sem0)

    return k(ts, tr, snd, rcv)


_SINCOEF = (6.28318531, -41.34170217, 81.60524536, -76.70576095,
            42.05737007, -15.08455476, 3.77595755, -0.61505996)


def _sin2pi(f):
    f = f - jnp.round(f)
    x2 = f * f
    p = jnp.float32(_SINCOEF[-1])
    for coef in _SINCOEF[-2::-1]:
        p = p * x2 + jnp.float32(coef)
    return f * p


def _edge_tc(ga, gb, rv, zc16, w1g16, w_abs, b1, w2, b2, wax1, bax1, wax2, bax2):
    be = 2000

    def body(ga_ref, gb_ref, rv_ref, zc_ref, w1g_ref, wab_ref, b1_ref, w2_ref,
             b2_ref, wax1_ref, bax1_ref, wax2_ref, bax2_ref,
             msg_ref, dlt_ref, att_ref):
        r = rv_ref[:, :3]
        a = jnp.sqrt(jnp.sum(r * r, axis=1, keepdims=True))
        rbf16 = _sin2pi(a * zc_ref[...]) / a
        geo = jnp.dot(rbf16, w1g_ref[...], preferred_element_type=_f32)
        geo = geo + a * wab_ref[...]
        h1 = jax.nn.silu(ga_ref[...] + gb_ref[...] + geo + b1_ref[...])
        m = jnp.dot(h1, w2_ref[...], preferred_element_type=_f32) + b2_ref[...]
        hax = jax.nn.silu(jnp.dot(m, wax1_ref[...], preferred_element_type=_f32)
                          + bax1_ref[...])
        out2 = jnp.dot(hax, wax2_ref[...], preferred_element_type=_f32) \
            + bax2_ref[...]
        att = jax.nn.sigmoid(out2[:, 0:1])
        px = out2[:, 1:2]
        delta = r * (px / (a + 1.0))
        msg_ref[...] = m * att
        dlt_ref[...] = jnp.concatenate([delta, jnp.zeros((be, 1), _f32)], axis=1)
        att_ref[...] = att

    full = lambda shape: pl.BlockSpec(shape, lambda i: (0, 0))
    return pl.pallas_call(
        body,
        grid=(E // be,),
        in_specs=[
            pl.BlockSpec((be, C), lambda i: (i, 0)),
            pl.BlockSpec((be, C), lambda i: (i, 0)),
            pl.BlockSpec((be, 4), lambda i: (i, 0)),
            full((1, 16)),
            full((16, C)),
            full((1, C)),
            full((1, C)),
            full((C, C)),
            full((1, C)),
            full((C, 2 * C)),
            full((1, 2 * C)),
            full((2 * C, C)),
            full((1, C)),
        ],
        out_specs=[
            pl.BlockSpec((be, C), lambda i: (i, 0)),
            pl.BlockSpec((be, 4), lambda i: (i, 0)),
            pl.BlockSpec((be, 1), lambda i: (i, 0)),
        ],
        out_shape=[
            jax.ShapeDtypeStruct((E, C), _f32),
            jax.ShapeDtypeStruct((E, 4), _f32),
            jax.ShapeDtypeStruct((E, 1), _f32),
        ],
    )(ga, gb, rv, zc16, w1g16, w_abs, b1, w2, b2, wax1, bax1, wax2, bax2)


def _sc_scatter_msg(msg, snd, zrows):
    mesh = plsc.VectorSubcoreMesh(core_axis_name="c", subcore_axis_name="s")

    @functools.partial(
        pl.kernel,
        out_type=jax.ShapeDtypeStruct((NC, NPAD, C), _f32),
        mesh=mesh,
        scratch_types=(
            pltpu.VMEM((SUB,), _i32),
            pltpu.VMEM((SUB,), _i32),
            pltpu.VMEM((SUB, C), _f32),
            pltpu.VMEM((SUB, C), _f32),
            pltpu.VMEM_SHARED((NPAD, C), _f32),
            pltpu.SemaphoreType.DMA,
            pltpu.SemaphoreType.DMA,
        ),
    )
    def k(msg_h, snd_h, z_h, outm_h, ix0, ix1, rows0, rows1, acc, sem0, sem1):
        c = lax.axis_index("c")
        s = lax.axis_index("s")
        wid = c * NS + s
        roff = pl.multiple_of(s * NPT, 8)
        pltpu.sync_copy(z_h.at[pl.ds(roff, NPT)], acc.at[pl.ds(roff, NPT)])
        plsc.subcore_barrier()
        base = pl.multiple_of(wid * EPW, 8)

        def fire(g, ix, rows, sem):
            off = pl.multiple_of(base + g * SUB, 8)
            pltpu.async_copy(snd_h.at[pl.ds(off, SUB)], ix, sem)
            pltpu.async_copy(msg_h.at[pl.ds(off, SUB)], rows, sem)

        def drain_scatter(g, ix, rows, sem):
            off = pl.multiple_of(base + g * SUB, 8)
            pltpu.make_async_copy(snd_h.at[pl.ds(off, SUB)], ix, sem).wait()
            pltpu.make_async_copy(msg_h.at[pl.ds(off, SUB)], rows, sem).wait()
            pltpu.sync_copy(rows, acc.at[ix], add=True)

        fire(0, ix0, rows0, sem0)

        def body(h, carry):
            g = h * 2
            fire(g + 1, ix1, rows1, sem1)
            drain_scatter(g, ix0, rows0, sem0)
            fire(g + 2, ix0, rows0, sem0)
            drain_scatter(g + 1, ix1, rows1, sem1)
            return carry

        lax.fori_loop(0, (NCH - 1) // 2, body, 0)
        drain_scatter(NCH - 1, ix0, rows0, sem0)
        plsc.subcore_barrier()
        pltpu.sync_copy(acc.at[pl.ds(roff, NPT)], outm_h.at[c, pl.ds(roff, NPT)])

    return k(msg, snd, zrows)


def _sc_scatter_delta(dvec, snd):
    mesh = plsc.VectorSubcoreMesh(core_axis_name="c", subcore_axis_name="s")

    @functools.partial(
        pl.kernel,
        out_type=jax.ShapeDtypeStruct((NW * N * 4,), _f32),
        mesh=mesh,
        scratch_types=(
            pltpu.VMEM((SUB,), _i32),
            pltpu.VMEM((SUB,), _i32),
            pltpu.VMEM((SUB * 4,), _f32),
            pltpu.VMEM((SUB * 4,), _f32),
            pltpu.VMEM((N * 4,), _f32),
            pltpu.SemaphoreType.DMA,
            pltpu.SemaphoreType.DMA,
        ),
        compiler_params=pltpu.CompilerParams(needs_layout_passes=False),
    )
    def k(dv_h, snd_h, outd_h, ix0, ix1, dbuf0, dbuf1, dacc, sem0, sem1):
        c = lax.axis_index("c")
        s = lax.axis_index("s")
        wid = c * NS + s
        z16 = jnp.zeros((16,), _f32)

        def zbody(g, carry):
            dacc[pl.ds(pl.multiple_of(g * 16, 8), 16)] = z16
            return carry

        lax.fori_loop(0, N * 4 // 16, zbody, 0)
        base = pl.multiple_of(wid * EPW, 8)
        lane = lax.iota(_i32, 16)

        def fire(g, ix, dbuf, sem):
            off = pl.multiple_of(base + g * SUB, 8)
            pltpu.async_copy(snd_h.at[pl.ds(off, SUB)], ix, sem)
            pltpu.async_copy(dv_h.at[pl.ds(off * 4, SUB * 4)], dbuf, sem)

        def drain_scatter(g, ix, dbuf, sem):
            off = pl.multiple_of(base + g * SUB, 8)
            pltpu.make_async_copy(snd_h.at[pl.ds(off, SUB)], ix, sem).wait()
            pltpu.make_async_copy(dv_h.at[pl.ds(off * 4, SUB * 4)], dbuf,
                                  sem).wait()
            for q in range(SUB // 16):
                s16 = ix[pl.ds(q * 16, 16)]
                src = q * 64 + lane * 4
                for comp in range(3):
                    vals = plsc.load_gather(dbuf, [src + comp])
                    plsc.addupdate_scatter(dacc, [s16 * 4 + comp], vals)

        fire(0, ix0, dbuf0, sem0)

        def body(h, carry):
            g = h * 2
            fire(g + 1, ix1, dbuf1, sem1)
            drain_scatter(g, ix0, dbuf0, sem0)
            fire(g + 2, ix0, dbuf0, sem0)
            drain_scatter(g + 1, ix1, dbuf1, sem1)
            return carry

        lax.fori_loop(0, (NCH - 1) // 2, body, 0)
        drain_scatter(NCH - 1, ix0, dbuf0, sem0)
        pltpu.sync_copy(dacc, outd_h.at[pl.ds(pl.multiple_of(wid * N * 4, 8),
                                              N * 4)])

    return k(dvec, snd)


def _node_tc(nf, coords, p0, p1, dparts, wn1a, wn1b, bn1, wn2, bn2):
    bn = 1000

    def body(nf_ref, co_ref, p0_ref, p1_ref, dp_ref, wa_ref, wb_ref, b1_ref,
             w2_ref, b2_ref, nfo_ref, coo_ref):
        m = p0_ref[...] + p1_ref[...]
        delta = jnp.sum(dp_ref[...], axis=0)[:, :3]
        nfb = nf_ref[...]
        h = jax.nn.silu(jnp.dot(nfb, wa_ref[...], preferred_element_type=_f32)
                        + jnp.dot(m, wb_ref[...], preferred_element_type=_f32)
                        + b1_ref[...])
        nfo_ref[...] = jnp.dot(h, w2_ref[...], preferred_element_type=_f32) \
            + b2_ref[...] + nfb
        coo_ref[...] = co_ref[...] + delta

    full = lambda shape: pl.BlockSpec(shape, lambda i: (0, 0))
    return pl.pallas_call(
        body,
        grid=(N // bn,),
        in_specs=[
            pl.BlockSpec((bn, C), lambda i: (i, 0)),
            pl.BlockSpec((bn, 3), lambda i: (i, 0)),
            pl.BlockSpec((bn, C), lambda i: (i, 0)),
            pl.BlockSpec((bn, C), lambda i: (i, 0)),
            pl.BlockSpec((NW, bn, 4), lambda i: (0, i, 0)),
            full((C, C)),
            full((C, C)),
            full((1, C)),
            full((C, C)),
            full((1, C)),
        ],
        out_specs=[
            pl.BlockSpec((bn, C), lambda i: (i, 0)),
            pl.BlockSpec((bn, 3), lambda i: (i, 0)),
        ],
        out_shape=[
            jax.ShapeDtypeStruct((N, C), _f32),
            jax.ShapeDtypeStruct((N, 3), _f32),
        ],
    )(nf, coords, p0, p1, dparts, wn1a, wn1b, bn1, wn2, bn2)


def kernel(node_feats, coordinates, edge_index, params):
    pe, pn, pa, px = params["phi_e"], params["phi_n"], params["att"], params["phi_x"]
    w1 = pe["W1"]
    w1i = w1[:C]
    w1j = w1[C:2 * C]
    w_abs = w1[2 * C:2 * C + 1]
    cut = params["bessel_cut_off"]
    amp = jnp.sqrt(2.0 / cut)
    zc16 = jnp.zeros((1, 16), _f32).at[0, :8].set(
        params["z_0k"] / (2.0 * jnp.pi * cut))
    w1g16 = jnp.zeros((16, C), _f32).at[:8].set(w1[2 * C + 1:] * amp)
    b1 = pe["b1"].reshape(1, C)
    b2 = pe["b2"].reshape(1, C)
    wax1 = jnp.concatenate([pa["W1"], px["W1"]], axis=1)
    bax1 = jnp.concatenate([pa["b1"], px["b1"]]).reshape(1, 2 * C)
    wax2 = jnp.zeros((2 * C, C), _f32)
    wax2 = wax2.at[:C, 0].set(pa["W2"][:, 0]).at[C:, 1].set(px["W2"][:, 0])
    bax2 = jnp.zeros((1, C), _f32)
    bax2 = bax2.at[0, 0].set(pa["b2"][0]).at[0, 1].set(px["b2"][0])
    wn1 = pn["W1"]
    wn1a, wn1b = wn1[:C], wn1[C:]
    bn1 = pn["b1"].reshape(1, C)
    bn2 = pn["b2"].reshape(1, C)

    snd = edge_index[0]
    rcv = edge_index[1]
    cxyz = coordinates.T

    ts, tr = _tables_tc(node_feats, w1i, w1j)
    rv = _sc_geo(cxyz[0], cxyz[1], cxyz[2], snd, rcv).reshape(E, 4)
    ga, gb = _sc_gather(ts, tr, snd, rcv)
    msg, dvec, att = _edge_tc(ga, gb, rv, zc16, w1g16, w_abs, b1, pe["W2"], b2,
                              wax1, bax1, wax2, bax2)
    partm = _sc_scatter_msg(msg, snd, jnp.zeros((NPAD, C), _f32))
    partd = _sc_scatter_delta(dvec.reshape(E * 4), snd)
    nf_new, co_new = _node_tc(node_feats, coordinates, partm[0], partm[1],
                              partd.reshape(NW, N, 4), wn1a, wn1b, bn1,
                              pn["W2"], bn2)
    return nf_new, co_new, att
